# baseline (speedup 1.0000x reference)
import functools

import jax
import jax.numpy as jnp
from jax import lax
from jax.experimental import pallas as pl
from jax.experimental.pallas import tpu as pltpu
from jax.experimental.pallas import tpu_sc as plsc

N_NODES = 10000
NR = 10240
NPT = NR // 16
F1 = 16
F2 = 32
G = 128
NBUF = 4
N_TILES = 16
GQ, GR = 2500 // N_TILES, 2500 % N_TILES
GTMAX = GQ + 1


def _mm_body(x_ref, w_ref, o_ref):
    n = x_ref.shape[0]
    o_ref[pl.ds(0, n), :] = jnp.dot(x_ref[...], w_ref[...],
                                    preferred_element_type=jnp.float32)
    o_ref[pl.ds(n, NR - n), :] = jnp.zeros((NR - n, F1), jnp.float32)


def _rsqrt16(d):
    i = lax.bitcast_convert_type(d, jnp.int32)
    y = lax.bitcast_convert_type(jnp.int32(0x5F3759DF) - (i >> 1), jnp.float32)
    for _ in range(3):
        y = y * (1.5 - 0.5 * d * y * y)
    return y


def _sc_body(GT, xw_ref, src_ref, dst_ref, b1_ref, w2_ref, b2_ref,
             wf1_ref, bf1_ref, wf2_ref, bf2_ref, out_ref,
             table, acc, deg, pool,
             sidx, didx, rowbuf, nbin, nbout, dv, ones,
             wb1, wW2, wb2, wWf1, wbf1, wWf2, wbf2,
             poolb, poolall, outb, gsem, ssem, dmasem):
    c = lax.axis_index("c")
    s = lax.axis_index("s")
    nbase = s * NPT
    extra = jnp.where(s < GR, 1, 0)
    gtw = GQ + extra
    gbase = s * GQ + jnp.minimum(s, GR)

    pltpu.sync_copy(src_ref.at[pl.ds(gbase, GQ)], sidx.at[pl.ds(0, GQ)])
    pltpu.sync_copy(dst_ref.at[pl.ds(gbase, GQ)], didx.at[pl.ds(0, GQ)])

    @pl.when(extra == 1)
    def _stage_tail():
        pltpu.sync_copy(src_ref.at[pl.ds(gbase + GQ, 1)],
                        sidx.at[pl.ds(GQ, 1)])
        pltpu.sync_copy(dst_ref.at[pl.ds(gbase + GQ, 1)],
                        didx.at[pl.ds(GQ, 1)])
    pltpu.sync_copy(b1_ref, wb1)
    pltpu.sync_copy(w2_ref, wW2)
    pltpu.sync_copy(b2_ref, wb2)
    pltpu.sync_copy(wf1_ref, wWf1)
    pltpu.sync_copy(bf1_ref, wbf1)
    pltpu.sync_copy(wf2_ref, wWf2)
    pltpu.sync_copy(bf2_ref, wbf2)
    for j in range(G // 16):
        ones[pl.ds(j * 16, 16)] = jnp.full((16,), 1.0, jnp.float32)

    def initdeg(j, carry):
        dv[pl.ds(j * 16, 16)] = jnp.full((16,), 1.0, jnp.float32)
        return carry
    lax.fori_loop(0, NPT // 16, initdeg, 0)
    pltpu.sync_copy(dv, deg.at[pl.ds(nbase, NPT)])
    plsc.subcore_barrier()

    scope_deg = jax.named_scope("deg_pass")
    scope_deg.__enter__()
    nblk = gtw // NBUF

    def dpass(b, carry):
        ds = [pltpu.async_copy(ones, deg.at[didx.at[b * NBUF + j]],
                               dmasem.at[j], add=True)
              for j in range(NBUF)]
        for d in ds:
            d.wait()
        return carry
    lax.fori_loop(0, nblk, dpass, 0)

    def dtail(g, carry):
        pltpu.sync_copy(ones, deg.at[didx.at[g]], add=True)
        return carry
    lax.fori_loop(nblk * NBUF, gtw, dtail, 0)
    plsc.subcore_barrier()
    scope_deg.__exit__(None, None, None)

    pltpu.sync_copy(deg.at[pl.ds(nbase, NPT)], dv)

    def newton(j, carry):
        dv[pl.ds(j * 16, 16)] = _rsqrt16(dv[pl.ds(j * 16, 16)])
        return carry
    lax.fori_loop(0, NPT // 16, newton, 0)
    pltpu.sync_copy(xw_ref.at[pl.ds(nbase, NPT)], nbin)

    def scale1(j, carry):
        dvec = dv[pl.ds(j * 16, 16)]
        for l in range(16):
            i = j * 16 + l
            nbout[i, :] = nbin[i, :] * dvec[l]
        return carry
    lax.fori_loop(0, NPT // 16, scale1, 0)
    pltpu.sync_copy(nbout, table.at[pl.ds(nbase, NPT)])
    pltpu.sync_copy(nbout, acc.at[pl.ds(nbase, NPT)])
    plsc.subcore_barrier()

    def epass_run():
        for j in range(NBUF):
            pltpu.async_copy(table.at[sidx.at[j]], rowbuf.at[j], gsem.at[j])

        def body(b, carry):
            gb = b * NBUF
            for j in range(NBUF):
                pltpu.make_async_copy(table.at[sidx.at[gb + j]],
                                      rowbuf.at[j], gsem.at[j]).wait()
                pltpu.async_copy(rowbuf.at[j], acc.at[didx.at[gb + j]],
                                 ssem.at[j], add=True)
            for j in range(NBUF):
                pltpu.make_async_copy(rowbuf.at[j], acc.at[didx.at[gb + j]],
                                      ssem.at[j]).wait()
                nxt = jnp.minimum(gb + NBUF + j, gtw - 1)
                pltpu.async_copy(table.at[sidx.at[nxt]], rowbuf.at[j],
                                 gsem.at[j])
            return carry
        lax.fori_loop(0, nblk, body, 0)
        for j in range(NBUF):
            pltpu.make_async_copy(table.at[sidx.at[0]], rowbuf.at[j],
                                  gsem.at[j]).wait()

        def tail(g, carry):
            pltpu.sync_copy(table.at[sidx.at[g]], rowbuf.at[0])
            pltpu.sync_copy(rowbuf.at[0], acc.at[didx.at[g]], add=True)
            return carry
        lax.fori_loop(nblk * NBUF, gtw, tail, 0)

    with jax.named_scope("edge_pass1"):
        epass_run()
        plsc.subcore_barrier()

    pltpu.sync_copy(acc.at[pl.ds(nbase, NPT)], nbin)
    b1v = wb1[...]

    def stage_c(j, carry):
        dvec = dv[pl.ds(j * 16, 16)]
        for l in range(16):
            i = j * 16 + l
            d = dvec[l]
            h = jnp.maximum(nbin[i, :] * d + b1v, 0.0)
            nbout[i, :] = h * d
        return carry
    lax.fori_loop(0, NPT // 16, stage_c, 0)
    pltpu.sync_copy(nbout, table.at[pl.ds(nbase, NPT)])
    pltpu.sync_copy(nbout, acc.at[pl.ds(nbase, NPT)])
    plsc.subcore_barrier()

    with jax.named_scope("edge_pass2"):
        epass_run()
        plsc.subcore_barrier()

    pltpu.sync_copy(acc.at[pl.ds(nbase, NPT)], nbin)
    b2lo = wb2[pl.ds(0, 16)]
    b2hi = wb2[pl.ds(16, 16)]

    def stage_e(j, carry):
        plo, phi = carry
        dvec = dv[pl.ds(j * 16, 16)]
        for l in range(16):
            i = j * 16 + l
            aggv = nbin[i, :] * dvec[l]
            lo = b2lo
            hi = b2hi
            for k in range(F1):
                a = aggv[k]
                lo = lo + a * wW2[k, pl.ds(0, 16)]
                hi = hi + a * wW2[k, pl.ds(16, 16)]
            lo = jnp.maximum(lo, 0.0)
            hi = jnp.maximum(hi, 0.0)
            m = jnp.where(nbase + i < N_NODES, 1.0, 0.0)
            plo = plo + lo * m
            phi = phi + hi * m
        return (plo, phi)

    zero16 = jnp.zeros((16,), jnp.float32)
    with jax.named_scope("stage_e"):
        plo, phi = lax.fori_loop(0, NPT // 16, stage_e, (zero16, zero16))
    poolb[0, pl.ds(0, 16)] = plo
    poolb[0, pl.ds(16, 16)] = phi
    pltpu.sync_copy(poolb, pool.at[pl.ds(s, 1)])
    plsc.subcore_barrier()

    @pl.when(jnp.logical_and(c == 0, s == 0))
    def _mlp():
        pltpu.sync_copy(pool, poolall)
        lo = poolall[0, pl.ds(0, 16)]
        hi = poolall[0, pl.ds(16, 16)]
        for r in range(1, N_TILES):
            lo = lo + poolall[r, pl.ds(0, 16)]
            hi = hi + poolall[r, pl.ds(16, 16)]
        inv_n = jnp.float32(1.0 / N_NODES)
        pooled = [lo * inv_n, hi * inv_n]
        h = [wbf1[pl.ds(16 * j, 16)] for j in range(4)]
        for k in range(F2):
            a = pooled[k // 16][k % 16]
            for j in range(4):
                h[j] = h[j] + a * wWf1[k, pl.ds(16 * j, 16)]
        h = [jnp.maximum(hj, 0.0) for hj in h]
        o = wbf2[...]
        for k in range(64):
            o = o + h[k // 16][k % 16] * wWf2[k, :]
        outb[0, :] = o
        pltpu.sync_copy(outb, out_ref)


def _make_sc_kernel(GT):
    mesh = plsc.VectorSubcoreMesh(core_axis_name="c", subcore_axis_name="s")
    f32 = jnp.float32
    return pl.kernel(
        functools.partial(_sc_body, GT),
        out_type=jax.ShapeDtypeStruct((1, 16), f32),
        mesh=mesh,
        compiler_params=pltpu.CompilerParams(use_tc_tiling_on_sc=False),
        scratch_types=[
            pltpu.VMEM_SHARED((NR, F1), f32),
            pltpu.VMEM_SHARED((NR, F1), f32),
            pltpu.VMEM_SHARED((NR,), f32),
            pltpu.VMEM_SHARED((N_TILES, F2), f32),
            pltpu.VMEM((GTMAX, G), jnp.int32),
            pltpu.VMEM((GTMAX, G), jnp.int32),
            pltpu.VMEM((NBUF, G, F1), f32),
            pltpu.VMEM((NPT, F1), f32),
            pltpu.VMEM((NPT, F1), f32),
            pltpu.VMEM((NPT,), f32),
            pltpu.VMEM((G,), f32),
            pltpu.VMEM((F1,), f32),
            pltpu.VMEM((F1, F2), f32),
            pltpu.VMEM((F2,), f32),
            pltpu.VMEM((F2, 64), f32),
            pltpu.VMEM((64,), f32),
            pltpu.VMEM((64, 16), f32),
            pltpu.VMEM((16,), f32),
            pltpu.VMEM((1, F2), f32),
            pltpu.VMEM((N_TILES, F2), f32),
            pltpu.VMEM((1, 16), f32),
            pltpu.SemaphoreType.DMA((NBUF,)),
            pltpu.SemaphoreType.DMA((NBUF,)),
            pltpu.SemaphoreType.DMA((NBUF,)),
        ],
    )


def kernel(x, edge_index, W1, b1, W2, b2, Wf1, bf1, Wf2, bf2):
    e = edge_index.shape[1]
    ei = edge_index.astype(jnp.int32)
    src2 = ei[0].reshape(e // G, G)
    dst2 = ei[1].reshape(e // G, G)

    xw_pad = pl.pallas_call(
        _mm_body,
        out_shape=jax.ShapeDtypeStruct((NR, F1), jnp.float32),
    )(x, W1)

    wf2p = jnp.pad(Wf2, ((0, 0), (0, 16 - Wf2.shape[1])))
    bf2p = jnp.pad(bf2, (0, 16 - bf2.shape[0]))

    out16 = _make_sc_kernel(0)(
        xw_pad, src2, dst2, b1, W2, b2, Wf1, bf1, wf2p, bf2p)
    return out16[:, :Wf2.shape[1]]


# speedup vs baseline: 1.2314x; 1.2314x over previous
import functools

import jax
import jax.numpy as jnp
from jax import lax
from jax.experimental import pallas as pl
from jax.experimental.pallas import tpu as pltpu
from jax.experimental.pallas import tpu_sc as plsc

N_NODES = 10000
NR = 10240
NPT = NR // 16
F1 = 16
F2 = 32
G = 128
NBUF = 8
N_TILES = 16
GQ, GR = 2500 // N_TILES, 2500 % N_TILES
GTMAX = GQ + 1


def _mm_body(x_ref, w_ref, o_ref):
    n = x_ref.shape[0]
    o_ref[pl.ds(0, n), :] = jnp.dot(x_ref[...], w_ref[...],
                                    preferred_element_type=jnp.float32)
    o_ref[pl.ds(n, NR - n), :] = jnp.zeros((NR - n, F1), jnp.float32)


def _rsqrt16(d):
    i = lax.bitcast_convert_type(d, jnp.int32)
    y = lax.bitcast_convert_type(jnp.int32(0x5F3759DF) - (i >> 1), jnp.float32)
    for _ in range(3):
        y = y * (1.5 - 0.5 * d * y * y)
    return y


def _sc_body(GT, xw_ref, ei_ref, b1_ref, w2_ref, b2_ref,
             wf1_ref, bf1_ref, wf2_ref, bf2_ref, out_ref,
             table, acc, deg, pool,
             sidx, didx, rowbuf, nbin, nbout, dv, ones,
             wb1, wW2, wb2, wWf1, wbf1, wWf2, wbf2,
             poolb, poolall, outb, gsem, ssem, dmasem, xwsem):
    c = lax.axis_index("c")
    s = lax.axis_index("s")
    nbase = s * NPT
    extra = jnp.where(s < GR, 1, 0)
    gtw = GQ + extra
    gbase = s * GQ + jnp.minimum(s, GR)

    pltpu.sync_copy(ei_ref.at[0, pl.ds(gbase, GQ)], sidx.at[pl.ds(0, GQ)])
    pltpu.sync_copy(ei_ref.at[1, pl.ds(gbase, GQ)], didx.at[pl.ds(0, GQ)])

    @pl.when(extra == 1)
    def _stage_tail():
        pltpu.sync_copy(ei_ref.at[0, pl.ds(gbase + GQ, 1)],
                        sidx.at[pl.ds(GQ, 1)])
        pltpu.sync_copy(ei_ref.at[1, pl.ds(gbase + GQ, 1)],
                        didx.at[pl.ds(GQ, 1)])
    xw_cp = pltpu.async_copy(xw_ref.at[pl.ds(nbase, NPT)], nbin, xwsem)
    pltpu.sync_copy(b1_ref, wb1)
    pltpu.sync_copy(w2_ref, wW2)
    pltpu.sync_copy(b2_ref, wb2)
    pltpu.sync_copy(wf1_ref, wWf1)
    pltpu.sync_copy(bf1_ref, wbf1)
    pltpu.sync_copy(wf2_ref, wWf2)
    pltpu.sync_copy(bf2_ref, wbf2)
    for j in range(G // 16):
        ones[pl.ds(j * 16, 16)] = jnp.full((16,), 1.0, jnp.float32)

    def initdeg(j, carry):
        dv[pl.ds(j * 16, 16)] = jnp.full((16,), 1.0, jnp.float32)
        return carry
    lax.fori_loop(0, NPT // 16, initdeg, 0)
    pltpu.sync_copy(dv, deg.at[pl.ds(nbase, NPT)])
    plsc.subcore_barrier()

    scope_deg = jax.named_scope("deg_pass")
    scope_deg.__enter__()
    nblk = gtw // NBUF

    def dpass(b, carry):
        ds = [pltpu.async_copy(ones, deg.at[didx.at[b * NBUF + j]],
                               dmasem.at[j], add=True)
              for j in range(NBUF)]
        for d in ds:
            d.wait()
        return carry
    lax.fori_loop(0, nblk, dpass, 0)

    def dtail(g, carry):
        pltpu.sync_copy(ones, deg.at[didx.at[g]], add=True)
        return carry
    lax.fori_loop(nblk * NBUF, gtw, dtail, 0)
    plsc.subcore_barrier()
    scope_deg.__exit__(None, None, None)

    pltpu.sync_copy(deg.at[pl.ds(nbase, NPT)], dv)

    def newton(j, carry):
        dv[pl.ds(j * 16, 16)] = _rsqrt16(dv[pl.ds(j * 16, 16)])
        return carry
    lax.fori_loop(0, NPT // 16, newton, 0)
    xw_cp.wait()

    def scale1(j, carry):
        dvec = dv[pl.ds(j * 16, 16)]
        for l in range(16):
            i = j * 16 + l
            nbout[i, :] = nbin[i, :] * dvec[l]
        return carry
    lax.fori_loop(0, NPT // 16, scale1, 0)
    pltpu.sync_copy(nbout, table.at[pl.ds(nbase, NPT)])
    pltpu.sync_copy(nbout, acc.at[pl.ds(nbase, NPT)])
    plsc.subcore_barrier()

    def epass_run():
        def body(b, carry):
            gb = b * NBUF
            gds = [pltpu.async_copy(table.at[sidx.at[gb + j]], rowbuf.at[j],
                                    gsem.at[j])
                   for j in range(NBUF)]
            sds = []
            for j in range(NBUF):
                gds[j].wait()
                sds.append(pltpu.async_copy(rowbuf.at[j],
                                            acc.at[didx.at[gb + j]],
                                            ssem.at[j], add=True))
            for d in sds:
                d.wait()
            return carry
        lax.fori_loop(0, nblk, body, 0)

        def tail(g, carry):
            pltpu.sync_copy(table.at[sidx.at[g]], rowbuf.at[0])
            pltpu.sync_copy(rowbuf.at[0], acc.at[didx.at[g]], add=True)
            return carry
        lax.fori_loop(nblk * NBUF, gtw, tail, 0)

    with jax.named_scope("edge_pass1"):
        epass_run()
        plsc.subcore_barrier()

    pltpu.sync_copy(acc.at[pl.ds(nbase, NPT)], nbin)
    b1v = wb1[...]

    def stage_c(j, carry):
        dvec = dv[pl.ds(j * 16, 16)]
        for l in range(16):
            i = j * 16 + l
            d = dvec[l]
            h = jnp.maximum(nbin[i, :] * d + b1v, 0.0)
            nbout[i, :] = h * d
        return carry
    lax.fori_loop(0, NPT // 16, stage_c, 0)
    pltpu.sync_copy(nbout, table.at[pl.ds(nbase, NPT)])
    pltpu.sync_copy(nbout, acc.at[pl.ds(nbase, NPT)])
    plsc.subcore_barrier()

    with jax.named_scope("edge_pass2"):
        epass_run()
        plsc.subcore_barrier()

    pltpu.sync_copy(acc.at[pl.ds(nbase, NPT)], nbin)
    b2lo = wb2[pl.ds(0, 16)]
    b2hi = wb2[pl.ds(16, 16)]

    def stage_e(j, carry):
        plo, phi = carry
        dvec = dv[pl.ds(j * 16, 16)]
        for l in range(16):
            i = j * 16 + l
            aggv = nbin[i, :] * dvec[l]
            lo = b2lo
            hi = b2hi
            for k in range(F1):
                a = aggv[k]
                lo = lo + a * wW2[k, pl.ds(0, 16)]
                hi = hi + a * wW2[k, pl.ds(16, 16)]
            lo = jnp.maximum(lo, 0.0)
            hi = jnp.maximum(hi, 0.0)
            m = jnp.where(nbase + i < N_NODES, 1.0, 0.0)
            plo = plo + lo * m
            phi = phi + hi * m
        return (plo, phi)

    zero16 = jnp.zeros((16,), jnp.float32)
    with jax.named_scope("stage_e"):
        plo, phi = lax.fori_loop(0, NPT // 16, stage_e, (zero16, zero16))
    poolb[0, pl.ds(0, 16)] = plo
    poolb[0, pl.ds(16, 16)] = phi
    pltpu.sync_copy(poolb, pool.at[pl.ds(s, 1)])
    plsc.subcore_barrier()

    @pl.when(jnp.logical_and(c == 0, s == 0))
    def _mlp():
        pltpu.sync_copy(pool, poolall)
        lo = poolall[0, pl.ds(0, 16)]
        hi = poolall[0, pl.ds(16, 16)]
        for r in range(1, N_TILES):
            lo = lo + poolall[r, pl.ds(0, 16)]
            hi = hi + poolall[r, pl.ds(16, 16)]
        inv_n = jnp.float32(1.0 / N_NODES)
        pooled = [lo * inv_n, hi * inv_n]
        h = [wbf1[pl.ds(16 * j, 16)] for j in range(4)]
        for k in range(F2):
            a = pooled[k // 16][k % 16]
            for j in range(4):
                h[j] = h[j] + a * wWf1[k, pl.ds(16 * j, 16)]
        h = [jnp.maximum(hj, 0.0) for hj in h]
        o = wbf2[...]
        for k in range(64):
            o = o + h[k // 16][k % 16] * wWf2[k, :]
        outb[0, :] = o
        pltpu.sync_copy(outb, out_ref)


def _make_sc_kernel(GT):
    mesh = plsc.VectorSubcoreMesh(core_axis_name="c", subcore_axis_name="s")
    f32 = jnp.float32
    return pl.kernel(
        functools.partial(_sc_body, GT),
        out_type=jax.ShapeDtypeStruct((1, 16), f32),
        mesh=mesh,
        compiler_params=pltpu.CompilerParams(use_tc_tiling_on_sc=False),
        scratch_types=[
            pltpu.VMEM_SHARED((NR, F1), f32),
            pltpu.VMEM_SHARED((NR, F1), f32),
            pltpu.VMEM_SHARED((NR,), f32),
            pltpu.VMEM_SHARED((N_TILES, F2), f32),
            pltpu.VMEM((GTMAX, G), jnp.int32),
            pltpu.VMEM((GTMAX, G), jnp.int32),
            pltpu.VMEM((NBUF, G, F1), f32),
            pltpu.VMEM((NPT, F1), f32),
            pltpu.VMEM((NPT, F1), f32),
            pltpu.VMEM((NPT,), f32),
            pltpu.VMEM((G,), f32),
            pltpu.VMEM((F1,), f32),
            pltpu.VMEM((F1, F2), f32),
            pltpu.VMEM((F2,), f32),
            pltpu.VMEM((F2, 64), f32),
            pltpu.VMEM((64,), f32),
            pltpu.VMEM((64, 16), f32),
            pltpu.VMEM((16,), f32),
            pltpu.VMEM((1, F2), f32),
            pltpu.VMEM((N_TILES, F2), f32),
            pltpu.VMEM((1, 16), f32),
            pltpu.SemaphoreType.DMA((NBUF,)),
            pltpu.SemaphoreType.DMA((NBUF,)),
            pltpu.SemaphoreType.DMA((NBUF,)),
            pltpu.SemaphoreType.DMA,
        ],
    )


def kernel(x, edge_index, W1, b1, W2, b2, Wf1, bf1, Wf2, bf2):
    e = edge_index.shape[1]
    ei2 = edge_index.astype(jnp.int32).reshape(2, e // G, G)

    xw_pad = pl.pallas_call(
        _mm_body,
        out_shape=jax.ShapeDtypeStruct((NR, F1), jnp.float32),
    )(x, W1)

    wf2p = jnp.pad(Wf2, ((0, 0), (0, 16 - Wf2.shape[1])))
    bf2p = jnp.pad(bf2, (0, 16 - bf2.shape[0]))

    out16 = _make_sc_kernel(0)(
        xw_pad, ei2, b1, W2, b2, Wf1, bf1, wf2p, bf2p)
    return out16[:, :Wf2.shape[1]]
